# baseline (speedup 1.0000x reference)
import functools

import jax
import jax.numpy as jnp
from jax import lax
from jax.experimental import pallas as pl
from jax.experimental.pallas import tpu as pltpu
from jax.experimental.pallas import tpu_sc as plsc

_L = 16


def _span_extract_sc(B, S, D, N, WDIM, WPAD, n_workers):
    total = B * N
    assert total % n_workers == 0
    per_w = total // n_workers
    C = min(per_w, 32)
    assert per_w % C == 0 and C % _L == 0
    n_chunks = per_w // C
    logN = N.bit_length() - 1
    assert N == (1 << logN)

    mesh = plsc.VectorSubcoreMesh(core_axis_name="c", subcore_axis_name="s")

    @functools.partial(
        pl.kernel,
        mesh=mesh,
        compiler_params=pltpu.CompilerParams(use_tc_tiling_on_sc=False),
        out_type=jax.ShapeDtypeStruct((total, 2 * D + WDIM), jnp.float32),
        scratch_types=[
            pltpu.VMEM((C,), jnp.int32),
            pltpu.VMEM((C,), jnp.int32),
            pltpu.VMEM((C,), jnp.int32),
            pltpu.VMEM((C,), jnp.int32),
            pltpu.VMEM((C,), jnp.int32),
            pltpu.VMEM((C, D), jnp.float32),
            pltpu.VMEM((C, D), jnp.float32),
            pltpu.VMEM((C, WPAD), jnp.float32),
            pltpu.SemaphoreType.DMA,
        ],
    )
    def k(seq_hbm, starts_hbm, ends_hbm, wemb_hbm, out_hbm,
          rs_v, re_v, sidx_v, eidx_v, widx_v, srow_v, erow_v, wrow_v, sem):
        wid = lax.axis_index("s") * 2 + lax.axis_index("c")
        lane = lax.iota(jnp.int32, _L)

        for c in range(n_chunks):
            span_base = wid * per_w + c * C
            pltpu.sync_copy(starts_hbm.at[pl.ds(span_base, C)], rs_v)
            pltpu.sync_copy(ends_hbm.at[pl.ds(span_base, C)], re_v)
            for j in range(C // _L):
                off = j * _L + lane
                s = rs_v[pl.ds(j * _L, _L)]
                e = re_v[pl.ds(j * _L, _L)]
                b = lax.shift_right_logical(span_base + off, logN)
                base = b * S
                sidx_v[pl.ds(j * _L, _L)] = base + s
                eidx_v[pl.ds(j * _L, _L)] = base + e
                widx_v[pl.ds(j * _L, _L)] = e - s
            cs = pltpu.async_copy(seq_hbm.at[sidx_v], srow_v, sem)
            ce = pltpu.async_copy(seq_hbm.at[eidx_v], erow_v, sem)
            cw = pltpu.async_copy(wemb_hbm.at[widx_v], wrow_v, sem)
            cs.wait()
            ce.wait()
            cw.wait()
            pltpu.sync_copy(srow_v, out_hbm.at[pl.ds(span_base, C), pl.ds(0, D)])
            pltpu.sync_copy(erow_v, out_hbm.at[pl.ds(span_base, C), pl.ds(D, D)])
            pltpu.sync_copy(wrow_v.at[:, pl.ds(0, WDIM)],
                            out_hbm.at[pl.ds(span_base, C), pl.ds(2 * D, WDIM)])

    return k


def kernel(sequence_tensor, span_indices, width_embedding):
    B, S, D = sequence_tensor.shape
    _, N, _ = span_indices.shape
    WDIM = width_embedding.shape[1]
    seq_flat = sequence_tensor.reshape(B * S, D)
    starts_flat = span_indices[:, :, 0].reshape(-1).astype(jnp.int32)
    ends_flat = span_indices[:, :, 1].reshape(-1).astype(jnp.int32)
    WPAD = ((WDIM + 127) // 128) * 128
    wemb = jnp.pad(width_embedding, ((0, 0), (0, WPAD - WDIM)))
    out = _span_extract_sc(B, S, D, N, WDIM, WPAD, 32)(
        seq_flat, starts_flat, ends_flat, wemb)
    return out.reshape(B, N, 2 * D + WDIM)
